# baseline (speedup 1.0000x reference)
import functools

import jax
import jax.numpy as jnp
import numpy as np
from jax import lax
from jax.experimental import pallas as pl
from jax.experimental.pallas import tpu as pltpu
from jax.experimental.pallas import tpu_sc as plsc

_LANES = 16


def _chunk_offsets(d):
    offs = list(range(0, d - _LANES + 1, _LANES))
    if d % _LANES:
        offs.append(d - _LANES)
    return offs


def _w_select(d):
    offs = _chunk_offsets(d)
    idx, msk = [], []
    covered = set()
    for off in offs:
        for j in range(_LANES):
            col = off + j
            if col < d and col not in covered:
                idx.append(col)
                msk.append(1.0)
                covered.add(col)
            else:
                idx.append(0)
                msk.append(0.0)
    return np.array(idx, np.int32), np.array(msk, np.float32), offs


@functools.cache
def _build(batch, seq, d):
    info = plsc.get_sparse_core_info()
    nc, ns = info.num_cores, info.num_subcores
    nw = nc * ns
    assert batch % nw == 0 and seq == 200 and d == 100
    rpw = batch // nw
    offs = _chunk_offsets(d)
    nch = len(offs)
    wpad = nch * _LANES
    dpad = 128
    s0 = 96
    s1 = seq - s0

    mesh = plsc.VectorSubcoreMesh(core_axis_name="c", subcore_axis_name="s")

    @functools.partial(
        pl.kernel,
        mesh=mesh,
        compiler_params=pltpu.CompilerParams(use_tc_tiling_on_sc=False,
                                             needs_layout_passes=False),
        out_type=jax.ShapeDtypeStruct((batch,), jnp.float32),
        scratch_types=[
            pltpu.VMEM((rpw, seq), jnp.int32),
            pltpu.VMEM((seq, dpad), jnp.float32),
            pltpu.VMEM((seq, dpad), jnp.float32),
            pltpu.VMEM((2 * wpad + _LANES,), jnp.float32),
            pltpu.VMEM((rpw,), jnp.float32),
            pltpu.SemaphoreType.DMA,
            pltpu.SemaphoreType.DMA,
        ],
    )
    def run(prem_hbm, hyp_hbm, table_hbm, w_hbm, out_hbm,
            idx_v, buf0, buf1, w_v, out_v, sem0, sem1):
        wid = lax.axis_index("s") * nc + lax.axis_index("c")
        base = wid * rpw
        pltpu.sync_copy(w_hbm, w_v)
        lanes = lax.iota(jnp.int32, _LANES)

        def issue(g, buf, sem):
            c0 = pltpu.async_copy(table_hbm.at[idx_v.at[g, pl.ds(0, s0)]],
                                  buf.at[pl.ds(0, s0)], sem)
            c1 = pltpu.async_copy(table_hbm.at[idx_v.at[g, pl.ds(s0, s1)]],
                                  buf.at[pl.ds(s0, s1)], sem)
            return c0, c1

        def pooled_dot(side, buf):
            init = tuple(jnp.full((_LANES,), -jnp.inf, jnp.float32)
                         for _ in range(nch))

            def mstep(r, carry):
                return tuple(jnp.maximum(c, buf[r, pl.ds(off, _LANES)])
                             for c, off in zip(carry, offs))

            maxes = lax.fori_loop(0, seq, mstep, init, unroll=4)
            acc = jnp.zeros((_LANES,), jnp.float32)
            for c in range(nch):
                acc = acc + maxes[c] * w_v[pl.ds(side * wpad + c * _LANES,
                                                 _LANES)]
            return jnp.sum(acc)

        for side, src_hbm in ((0, prem_hbm), (1, hyp_hbm)):
            pltpu.sync_copy(src_hbm.at[pl.ds(base, rpw)], idx_v)

            def gstep(gg, acc, side=side):
                t0, t1 = 2 * gg, 2 * gg + 1
                h0 = issue(t0, buf0, sem0)
                h1 = issue(t1, buf1, sem1)
                for h in h0:
                    h.wait()
                d0 = pooled_dot(side, buf0)
                for h in h1:
                    h.wait()
                d1 = pooled_dot(side, buf1)
                acc = jnp.where(lanes == t0 % _LANES, d0, acc)
                acc = jnp.where(lanes == t1 % _LANES, d1, acc)

                @pl.when(t1 % _LANES == _LANES - 1)
                def _():
                    sl = pl.ds((t1 // _LANES) * _LANES, _LANES)
                    if side == 0:
                        out_v[sl] = acc
                    else:
                        out_v[sl] = out_v[sl] + acc

                return acc

            lax.fori_loop(0, rpw // 2, gstep,
                          jnp.zeros((_LANES,), jnp.float32))

        bvec = w_v[pl.ds(2 * wpad, _LANES)]
        for i in range(rpw // _LANES):
            x = out_v[pl.ds(i * _LANES, _LANES)] + bvec
            out_v[pl.ds(i * _LANES, _LANES)] = 1.0 / (1.0 + jnp.exp(-x))
        pltpu.sync_copy(out_v, out_hbm.at[pl.ds(base, rpw)])

    return run


def kernel(premise, hypothesis, table, W, b):
    batch, seq = premise.shape
    vocab, d = table.shape
    sel, msk, _ = _w_select(d)
    wp = W[0, :d][sel] * msk
    wh = W[0, d:][sel] * msk
    w_full = jnp.concatenate([wp, wh, jnp.broadcast_to(b, (_LANES,))])
    table_pad = jax.lax.optimization_barrier(
        jnp.pad(table, ((0, 0), (0, 128 - d))))
    run = _build(batch, seq, d)
    return run(premise, hypothesis, table_pad, w_full)


# speedup vs baseline: 1.0022x; 1.0022x over previous
import functools

import jax
import jax.numpy as jnp
import numpy as np
from jax import lax
from jax.experimental import pallas as pl
from jax.experimental.pallas import tpu as pltpu
from jax.experimental.pallas import tpu_sc as plsc

_LANES = 16


def _chunk_offsets(d):
    offs = list(range(0, d - _LANES + 1, _LANES))
    if d % _LANES:
        offs.append(d - _LANES)
    return offs


def _w_select(d):
    offs = _chunk_offsets(d)
    idx, msk = [], []
    covered = set()
    for off in offs:
        for j in range(_LANES):
            col = off + j
            if col < d and col not in covered:
                idx.append(col)
                msk.append(1.0)
                covered.add(col)
            else:
                idx.append(0)
                msk.append(0.0)
    return np.array(idx, np.int32), np.array(msk, np.float32), offs


@functools.cache
def _build(batch, seq, d):
    info = plsc.get_sparse_core_info()
    nc, ns = info.num_cores, info.num_subcores
    nw = nc * ns
    assert batch % nw == 0 and seq == 200 and d == 100
    rpw = batch // nw
    offs = _chunk_offsets(d)
    nch = len(offs)
    wpad = nch * _LANES
    dpad = 128
    s0 = 96
    s1 = seq - s0

    mesh = plsc.VectorSubcoreMesh(core_axis_name="c", subcore_axis_name="s")

    @functools.partial(
        pl.kernel,
        mesh=mesh,
        compiler_params=pltpu.CompilerParams(use_tc_tiling_on_sc=True,
                                             needs_layout_passes=False),
        out_type=jax.ShapeDtypeStruct((batch,), jnp.float32),
        scratch_types=[
            pltpu.VMEM((rpw * seq,), jnp.int32),
            pltpu.VMEM((seq, dpad), jnp.float32),
            pltpu.VMEM((seq, dpad), jnp.float32),
            pltpu.VMEM((2 * wpad + _LANES,), jnp.float32),
            pltpu.VMEM((rpw,), jnp.float32),
            pltpu.SemaphoreType.DMA,
            pltpu.SemaphoreType.DMA,
        ],
    )
    def run(prem_hbm, hyp_hbm, table_hbm, w_hbm, out_hbm,
            idx_v, buf0, buf1, w_v, out_v, sem0, sem1):
        wid = lax.axis_index("s") * nc + lax.axis_index("c")
        base = wid * rpw
        pltpu.sync_copy(w_hbm, w_v)
        lanes = lax.iota(jnp.int32, _LANES)

        def issue(g, buf, sem):
            c0 = pltpu.async_copy(
                table_hbm.at[idx_v.at[pl.ds(g * seq, s0)]],
                buf.at[pl.ds(0, s0)], sem)
            c1 = pltpu.async_copy(
                table_hbm.at[idx_v.at[pl.ds(g * seq + s0, s1)]],
                buf.at[pl.ds(s0, s1)], sem)
            return c0, c1

        def pooled_dot(side, buf):
            init = tuple(jnp.full((_LANES,), -jnp.inf, jnp.float32)
                         for _ in range(nch))

            def mstep(r, carry):
                return tuple(jnp.maximum(c, buf[r, pl.ds(off, _LANES)])
                             for c, off in zip(carry, offs))

            maxes = lax.fori_loop(0, seq, mstep, init, unroll=4)
            acc = jnp.zeros((_LANES,), jnp.float32)
            for c in range(nch):
                acc = acc + maxes[c] * w_v[pl.ds(side * wpad + c * _LANES,
                                                 _LANES)]
            return jnp.sum(acc)

        for side, src_hbm in ((0, prem_hbm), (1, hyp_hbm)):
            pltpu.sync_copy(src_hbm.at[pl.ds(base * seq, rpw * seq)], idx_v)

            def gstep(gg, acc, side=side):
                t0, t1 = 2 * gg, 2 * gg + 1
                h0 = issue(t0, buf0, sem0)
                h1 = issue(t1, buf1, sem1)
                for h in h0:
                    h.wait()
                d0 = pooled_dot(side, buf0)
                for h in h1:
                    h.wait()
                d1 = pooled_dot(side, buf1)
                acc = jnp.where(lanes == t0 % _LANES, d0, acc)
                acc = jnp.where(lanes == t1 % _LANES, d1, acc)

                @pl.when(t1 % _LANES == _LANES - 1)
                def _():
                    sl = pl.ds((t1 // _LANES) * _LANES, _LANES)
                    if side == 0:
                        out_v[sl] = acc
                    else:
                        out_v[sl] = out_v[sl] + acc

                return acc

            lax.fori_loop(0, rpw // 2, gstep,
                          jnp.zeros((_LANES,), jnp.float32))

        bvec = w_v[pl.ds(2 * wpad, _LANES)]
        for i in range(rpw // _LANES):
            x = out_v[pl.ds(i * _LANES, _LANES)] + bvec
            out_v[pl.ds(i * _LANES, _LANES)] = 1.0 / (1.0 + jnp.exp(-x))
        pltpu.sync_copy(out_v, out_hbm.at[pl.ds(base, rpw)])

    return run


def kernel(premise, hypothesis, table, W, b):
    batch, seq = premise.shape
    vocab, d = table.shape
    sel, msk, _ = _w_select(d)
    wp = W[0, :d][sel] * msk
    wh = W[0, d:][sel] * msk
    w_full = jnp.concatenate([wp, wh, jnp.broadcast_to(b, (_LANES,))])
    table_pad = jnp.pad(table, ((0, 0), (0, 128 - d)))
    run = _build(batch, seq, d)
    return run(premise.reshape(-1), hypothesis.reshape(-1),
               table_pad, w_full)


# speedup vs baseline: 1.8471x; 1.8430x over previous
import functools

import jax
import jax.numpy as jnp
import numpy as np
from jax import lax
from jax.experimental import pallas as pl
from jax.experimental.pallas import tpu as pltpu
from jax.experimental.pallas import tpu_sc as plsc

_LANES = 16


def _chunk_offsets(d):
    offs = list(range(0, d - _LANES + 1, _LANES))
    if d % _LANES:
        offs.append(d - _LANES)
    return offs


def _w_select(d):
    offs = _chunk_offsets(d)
    idx, msk = [], []
    covered = set()
    for off in offs:
        for j in range(_LANES):
            col = off + j
            if col < d and col not in covered:
                idx.append(col)
                msk.append(1.0)
                covered.add(col)
            else:
                idx.append(0)
                msk.append(0.0)
    return np.array(idx, np.int32), np.array(msk, np.float32), offs


@functools.cache
def _build_pad(vocab, d, dpad, blk):
    def body(x_ref, o_ref):
        o_ref[:, :d] = x_ref[...]
        o_ref[:, d:] = jnp.zeros((blk, dpad - d), jnp.float32)

    return pl.pallas_call(
        body,
        grid=(vocab // blk,),
        in_specs=[pl.BlockSpec((blk, d), lambda i: (i, 0))],
        out_specs=pl.BlockSpec((blk, dpad), lambda i: (i, 0)),
        out_shape=jax.ShapeDtypeStruct((vocab, dpad), jnp.float32),
    )


@functools.cache
def _build(batch, seq, d):
    info = plsc.get_sparse_core_info()
    nc, ns = info.num_cores, info.num_subcores
    nw = nc * ns
    assert batch % nw == 0 and seq == 200 and d == 100
    rpw = batch // nw
    offs = _chunk_offsets(d)
    nch = len(offs)
    wpad = nch * _LANES
    dpad = 128
    s0 = 96
    s1 = seq - s0

    mesh = plsc.VectorSubcoreMesh(core_axis_name="c", subcore_axis_name="s")

    @functools.partial(
        pl.kernel,
        mesh=mesh,
        compiler_params=pltpu.CompilerParams(use_tc_tiling_on_sc=True,
                                             needs_layout_passes=False),
        out_type=jax.ShapeDtypeStruct((batch,), jnp.float32),
        scratch_types=[
            pltpu.VMEM((rpw * seq,), jnp.int32),
            pltpu.VMEM((seq, dpad), jnp.float32),
            pltpu.VMEM((seq, dpad), jnp.float32),
            pltpu.VMEM((2 * wpad + _LANES,), jnp.float32),
            pltpu.VMEM((rpw,), jnp.float32),
            pltpu.SemaphoreType.DMA,
            pltpu.SemaphoreType.DMA,
        ],
    )
    def run(prem_hbm, hyp_hbm, table_hbm, w_hbm, out_hbm,
            idx_v, buf0, buf1, w_v, out_v, sem0, sem1):
        wid = lax.axis_index("s") * nc + lax.axis_index("c")
        base = wid * rpw
        pltpu.sync_copy(w_hbm, w_v)
        lanes = lax.iota(jnp.int32, _LANES)

        def issue(g, buf, sem):
            c0 = pltpu.async_copy(
                table_hbm.at[idx_v.at[pl.ds(g * seq, s0)]],
                buf.at[pl.ds(0, s0)], sem)
            c1 = pltpu.async_copy(
                table_hbm.at[idx_v.at[pl.ds(g * seq + s0, s1)]],
                buf.at[pl.ds(s0, s1)], sem)
            return c0, c1

        def pooled_dot(side, buf):
            init = tuple(jnp.full((_LANES,), -jnp.inf, jnp.float32)
                         for _ in range(nch))

            def mstep(r, carry):
                return tuple(jnp.maximum(c, buf[r, pl.ds(off, _LANES)])
                             for c, off in zip(carry, offs))

            maxes = lax.fori_loop(0, seq, mstep, init, unroll=4)
            acc = jnp.zeros((_LANES,), jnp.float32)
            for c in range(nch):
                acc = acc + maxes[c] * w_v[pl.ds(side * wpad + c * _LANES,
                                                 _LANES)]
            return jnp.sum(acc)

        for side, src_hbm in ((0, prem_hbm), (1, hyp_hbm)):
            pltpu.sync_copy(src_hbm.at[pl.ds(base * seq, rpw * seq)], idx_v)

            def gstep(gg, acc, side=side):
                t0, t1 = 2 * gg, 2 * gg + 1
                h0 = issue(t0, buf0, sem0)
                h1 = issue(t1, buf1, sem1)
                for h in h0:
                    h.wait()
                d0 = pooled_dot(side, buf0)
                for h in h1:
                    h.wait()
                d1 = pooled_dot(side, buf1)
                acc = jnp.where(lanes == t0 % _LANES, d0, acc)
                acc = jnp.where(lanes == t1 % _LANES, d1, acc)

                @pl.when(t1 % _LANES == _LANES - 1)
                def _():
                    sl = pl.ds((t1 // _LANES) * _LANES, _LANES)
                    if side == 0:
                        out_v[sl] = acc
                    else:
                        out_v[sl] = out_v[sl] + acc

                return acc

            lax.fori_loop(0, rpw // 2, gstep,
                          jnp.zeros((_LANES,), jnp.float32))

        bvec = w_v[pl.ds(2 * wpad, _LANES)]
        for i in range(rpw // _LANES):
            x = out_v[pl.ds(i * _LANES, _LANES)] + bvec
            out_v[pl.ds(i * _LANES, _LANES)] = 1.0 / (1.0 + jnp.exp(-x))
        pltpu.sync_copy(out_v, out_hbm.at[pl.ds(base, rpw)])

    return run


def kernel(premise, hypothesis, table, W, b):
    batch, seq = premise.shape
    vocab, d = table.shape
    sel, msk, _ = _w_select(d)
    wp = W[0, :d][sel] * msk
    wh = W[0, d:][sel] * msk
    w_full = jnp.concatenate([wp, wh, jnp.broadcast_to(b, (_LANES,))])
    table_pad = _build_pad(vocab, d, 128, 4000)(table)
    run = _build(batch, seq, d)
    return run(premise.reshape(-1), hypothesis.reshape(-1),
               table_pad, w_full)
